# baseline (device time: 67092 ns/iter reference)
import jax
import jax.numpy as jnp
from jax import lax
from jax.experimental import pallas as pl
from jax.experimental.pallas import tpu as pltpu

N_DEV = 4
SQ = 512
HA = SQ // 2
D = 1024
N_HEADS = 8
DH = 128
SCALE = 0.08838834764831843


def kernel(x, Wq, Wo, Wk, Wv):
    def body(x_ref, wq_ref, wk_ref, wv_ref, wo_ref, out_ref,
             xg_ref, acc0_ref, obr_ref, obl_ref, obo_ref,
             rsdr_ref, rsdl_ref, fina_ref, finb_ref,
             attn_ref, wqb_ref, wkb_ref, wvb_ref, wob_ref,
             send_sems, recv_sems):
        my = lax.axis_index("i")
        left = lax.rem(my + (N_DEV - 1), N_DEV)
        right = lax.rem(my + 1, N_DEV)
        opp = lax.rem(my + 2, N_DEV)

        barrier_sem = pltpu.get_barrier_semaphore()
        for nbr in (left, right, opp):
            pl.semaphore_signal(
                barrier_sem, inc=1,
                device_id=(nbr,), device_id_type=pl.DeviceIdType.MESH,
            )
        pl.semaphore_wait(barrier_sem, 3)

        rowsA = pl.ds(0, HA)
        rowsB = pl.ds(HA, HA)

        def rdma(idx, src, dst, to):
            return pltpu.make_async_remote_copy(
                src_ref=src, dst_ref=dst,
                send_sem=send_sems.at[idx], recv_sem=recv_sems.at[idx],
                device_id=(to,), device_id_type=pl.DeviceIdType.MESH,
            )

        def qkv(s):
            xs = xg_ref[s]
            q = (jnp.dot(xs, wqb_ref[...], preferred_element_type=jnp.float32)
                 * SCALE).astype(jnp.bfloat16)
            k = jnp.dot(xs, wkb_ref[...],
                        preferred_element_type=jnp.float32).astype(jnp.bfloat16)
            v = jnp.dot(xs, wvb_ref[...],
                        preferred_element_type=jnp.float32).astype(jnp.bfloat16)
            return q, k, v

        def attn_rows(q, k, v, rows):
            for hh in range(N_HEADS):
                sl = slice(hh * DH, (hh + 1) * DH)
                scores = lax.dot_general(
                    q[rows, sl], k[:, sl],
                    (((1,), (1,)), ((), ())),
                    preferred_element_type=jnp.float32,
                )
                p = jnp.exp(scores)
                l = jnp.sum(p, axis=-1, keepdims=True)
                o = jnp.dot(p.astype(jnp.bfloat16), v[:, sl],
                            preferred_element_type=jnp.float32)
                attn_ref[rows, sl] = (o / l).astype(jnp.bfloat16)
            return jnp.dot(attn_ref[rows, :], wob_ref[...],
                           preferred_element_type=jnp.float32)

        def attn_part(s):
            q, k, v = qkv(s)
            return attn_rows(q, k, v, slice(None))

        xg_ref[0] = x_ref[0].astype(jnp.bfloat16)
        agr1 = rdma(0, xg_ref.at[0], xg_ref.at[1], right)
        agl1 = rdma(1, xg_ref.at[0], xg_ref.at[3], left)
        agr1.start()
        agl1.start()

        wqb_ref[...] = wq_ref[...].astype(jnp.bfloat16)
        wkb_ref[...] = wk_ref[...].astype(jnp.bfloat16)
        wvb_ref[...] = wv_ref[...].astype(jnp.bfloat16)
        wob_ref[...] = wo_ref[...].astype(jnp.bfloat16)

        acc0_ref[...] = attn_part(0)

        agr1.wait_recv()
        agda = rdma(2, xg_ref.at[1, rowsA], xg_ref.at[2, rowsA], right)
        agda.start()
        agl1.wait_recv()
        agdb = rdma(3, xg_ref.at[3, rowsB], xg_ref.at[2, rowsB], left)
        agdb.start()

        q1, k1, v1 = qkv(1)
        obl_ref[rowsA] = attn_rows(q1, k1, v1,
                                   slice(0, HA)).astype(jnp.bfloat16)
        drla = rdma(4, obl_ref.at[rowsA], rsdl_ref.at[rowsA], left)
        drla.start()

        agda.wait_recv()
        agdb.wait_recv()
        obo_ref[...] = attn_part(2).astype(jnp.bfloat16)
        ohda = rdma(8, obo_ref.at[rowsA], fina_ref, opp)
        ohdb = rdma(9, obo_ref.at[rowsB], finb_ref, opp)
        ohda.start()
        ohdb.start()

        q3, k3, v3 = qkv(3)
        obr_ref[rowsB] = attn_rows(q3, k3, v3,
                                   slice(HA, SQ)).astype(jnp.bfloat16)
        drrb = rdma(7, obr_ref.at[rowsB], rsdr_ref.at[rowsB], right)
        drrb.start()
        obl_ref[rowsB] = attn_rows(q1, k1, v1,
                                   slice(HA, SQ)).astype(jnp.bfloat16)
        drlb = rdma(5, obl_ref.at[rowsB], rsdl_ref.at[rowsB], left)
        drlb.start()
        obr_ref[rowsA] = attn_rows(q3, k3, v3,
                                   slice(0, HA)).astype(jnp.bfloat16)
        drra = rdma(6, obr_ref.at[rowsA], rsdr_ref.at[rowsA], right)
        drra.start()

        drrb.wait_recv()
        drlb.wait_recv()
        ohdb.wait_recv()
        out_ref[0, HA:, :] = (acc0_ref[HA:, :]
                              + rsdr_ref[HA:, :].astype(jnp.float32)
                              + rsdl_ref[HA:, :].astype(jnp.float32)
                              + finb_ref[...].astype(jnp.float32)
                              ).astype(jnp.bfloat16)
        drra.wait_recv()
        drla.wait_recv()
        ohda.wait_recv()
        out_ref[0, :HA, :] = (acc0_ref[:HA, :]
                              + rsdr_ref[:HA, :].astype(jnp.float32)
                              + rsdl_ref[:HA, :].astype(jnp.float32)
                              + fina_ref[...].astype(jnp.float32)
                              ).astype(jnp.bfloat16)

        for r in (agr1, agl1, agda, agdb, drla, drlb, drra, drrb,
                  ohda, ohdb):
            r.wait_send()

    return pl.pallas_call(
        body,
        out_shape=jax.ShapeDtypeStruct((1, SQ, D), jnp.bfloat16),
        in_specs=[pl.BlockSpec(memory_space=pltpu.VMEM)] * 5,
        out_specs=pl.BlockSpec(memory_space=pltpu.VMEM),
        scratch_shapes=[
            pltpu.VMEM((N_DEV, SQ, D), jnp.bfloat16),
            pltpu.VMEM((SQ, D), jnp.float32),
            pltpu.VMEM((SQ, D), jnp.bfloat16),
            pltpu.VMEM((SQ, D), jnp.bfloat16),
            pltpu.VMEM((SQ, D), jnp.bfloat16),
            pltpu.VMEM((SQ, D), jnp.bfloat16),
            pltpu.VMEM((SQ, D), jnp.bfloat16),
            pltpu.VMEM((HA, D), jnp.bfloat16),
            pltpu.VMEM((HA, D), jnp.bfloat16),
            pltpu.VMEM((SQ, D), jnp.bfloat16),
            pltpu.VMEM((D, D), jnp.bfloat16),
            pltpu.VMEM((D, D), jnp.bfloat16),
            pltpu.VMEM((D, D), jnp.bfloat16),
            pltpu.VMEM((D, D), jnp.bfloat16),
            pltpu.SemaphoreType.DMA((10,)),
            pltpu.SemaphoreType.DMA((10,)),
        ],
        compiler_params=pltpu.CompilerParams(collective_id=0),
    )(x, Wq, Wk, Wv, Wo)


# device time: 65617 ns/iter; 1.0225x vs baseline; 1.0225x over previous
import jax
import jax.numpy as jnp
from jax import lax
from jax.experimental import pallas as pl
from jax.experimental.pallas import tpu as pltpu

N_DEV = 4
SQ = 512
HA = SQ // 2
D = 1024
N_HEADS = 8
DH = 128
SCALE = 0.08838834764831843 * 1.4426950408889634


def kernel(x, Wq, Wo, Wk, Wv):
    def body(x_ref, wq_ref, wk_ref, wv_ref, wo_ref, out_ref,
             xg_ref, acc0_ref, obr_ref, obl_ref, obo_ref,
             rsdr_ref, rsdl_ref, fina_ref, finb_ref,
             attn_ref, wqb_ref, wkb_ref, wvb_ref, wob_ref,
             send_sems, recv_sems):
        my = lax.axis_index("i")
        left = lax.rem(my + (N_DEV - 1), N_DEV)
        right = lax.rem(my + 1, N_DEV)
        opp = lax.rem(my + 2, N_DEV)

        barrier_sem = pltpu.get_barrier_semaphore()
        for nbr in (left, right, opp):
            pl.semaphore_signal(
                barrier_sem, inc=1,
                device_id=(nbr,), device_id_type=pl.DeviceIdType.MESH,
            )
        pl.semaphore_wait(barrier_sem, 3)

        rowsA = pl.ds(0, HA)
        rowsB = pl.ds(HA, HA)

        def rdma(idx, src, dst, to):
            return pltpu.make_async_remote_copy(
                src_ref=src, dst_ref=dst,
                send_sem=send_sems.at[idx], recv_sem=recv_sems.at[idx],
                device_id=(to,), device_id_type=pl.DeviceIdType.MESH,
            )

        def qkv(s):
            xs = xg_ref[s]
            q = (jnp.dot(xs, wqb_ref[...], preferred_element_type=jnp.float32)
                 * SCALE).astype(jnp.bfloat16)
            k = jnp.dot(xs, wkb_ref[...],
                        preferred_element_type=jnp.float32).astype(jnp.bfloat16)
            v = jnp.dot(xs, wvb_ref[...],
                        preferred_element_type=jnp.float32).astype(jnp.bfloat16)
            return q, k, v

        def attn_rows(q, k, v, rows):
            for hh in range(N_HEADS):
                sl = slice(hh * DH, (hh + 1) * DH)
                scores = lax.dot_general(
                    q[rows, sl], k[:, sl],
                    (((1,), (1,)), ((), ())),
                    preferred_element_type=jnp.float32,
                )
                p = jnp.exp2(scores)
                rl = 1.0 / jnp.sum(p, axis=-1, keepdims=True)
                o = jnp.dot(p.astype(jnp.bfloat16), v[:, sl],
                            preferred_element_type=jnp.float32)
                attn_ref[rows, sl] = (o * rl).astype(jnp.bfloat16)
            return jnp.dot(attn_ref[rows, :], wob_ref[...],
                           preferred_element_type=jnp.float32)

        def attn_part(s):
            q, k, v = qkv(s)
            return attn_rows(q, k, v, slice(None))

        xg_ref[0] = x_ref[0].astype(jnp.bfloat16)
        agr1a = rdma(0, xg_ref.at[0, rowsA], xg_ref.at[1, rowsA], right)
        agr1b = rdma(1, xg_ref.at[0, rowsB], xg_ref.at[1, rowsB], right)
        agl1b = rdma(2, xg_ref.at[0, rowsB], xg_ref.at[3, rowsB], left)
        agl1a = rdma(3, xg_ref.at[0, rowsA], xg_ref.at[3, rowsA], left)
        agr1a.start()
        agr1b.start()
        agl1b.start()
        agl1a.start()

        wqb_ref[...] = wq_ref[...].astype(jnp.bfloat16)
        wkb_ref[...] = wk_ref[...].astype(jnp.bfloat16)
        wvb_ref[...] = wv_ref[...].astype(jnp.bfloat16)

        q0, k0, v0 = qkv(0)
        wob_ref[...] = wo_ref[...].astype(jnp.bfloat16)

        agr1a.wait_recv()
        agda = rdma(4, xg_ref.at[1, rowsA], xg_ref.at[2, rowsA], right)
        agda.start()
        agl1b.wait_recv()
        agdb = rdma(5, xg_ref.at[3, rowsB], xg_ref.at[2, rowsB], left)
        agdb.start()

        acc0_ref[...] = attn_rows(q0, k0, v0, slice(None))

        agr1b.wait_recv()
        q1, k1, v1 = qkv(1)
        obl_ref[rowsA] = attn_rows(q1, k1, v1,
                                   slice(0, HA)).astype(jnp.bfloat16)
        drla = rdma(6, obl_ref.at[rowsA], rsdl_ref.at[rowsA], left)
        drla.start()

        agda.wait_recv()
        agdb.wait_recv()
        obo_ref[...] = attn_part(2).astype(jnp.bfloat16)
        ohda = rdma(10, obo_ref.at[rowsA], fina_ref, opp)
        ohdb = rdma(11, obo_ref.at[rowsB], finb_ref, opp)
        ohda.start()
        ohdb.start()

        agl1a.wait_recv()
        q3, k3, v3 = qkv(3)
        obr_ref[rowsB] = attn_rows(q3, k3, v3,
                                   slice(HA, SQ)).astype(jnp.bfloat16)
        drrb = rdma(9, obr_ref.at[rowsB], rsdr_ref.at[rowsB], right)
        drrb.start()
        obl_ref[rowsB] = attn_rows(q1, k1, v1,
                                   slice(HA, SQ)).astype(jnp.bfloat16)
        drlb = rdma(7, obl_ref.at[rowsB], rsdl_ref.at[rowsB], left)
        drlb.start()
        obr_ref[rowsA] = attn_rows(q3, k3, v3,
                                   slice(0, HA)).astype(jnp.bfloat16)
        drra = rdma(8, obr_ref.at[rowsA], rsdr_ref.at[rowsA], right)
        drra.start()

        drrb.wait_recv()
        drlb.wait_recv()
        ohdb.wait_recv()
        out_ref[0, HA:, :] = (acc0_ref[HA:, :]
                              + rsdr_ref[HA:, :].astype(jnp.float32)
                              + rsdl_ref[HA:, :].astype(jnp.float32)
                              + finb_ref[...].astype(jnp.float32)
                              ).astype(jnp.bfloat16)
        drra.wait_recv()
        drla.wait_recv()
        ohda.wait_recv()
        out_ref[0, :HA, :] = (acc0_ref[:HA, :]
                              + rsdr_ref[:HA, :].astype(jnp.float32)
                              + rsdl_ref[:HA, :].astype(jnp.float32)
                              + fina_ref[...].astype(jnp.float32)
                              ).astype(jnp.bfloat16)

        for r in (agr1a, agr1b, agl1a, agl1b, agda, agdb,
                  drla, drlb, drra, drrb, ohda, ohdb):
            r.wait_send()

    return pl.pallas_call(
        body,
        out_shape=jax.ShapeDtypeStruct((1, SQ, D), jnp.bfloat16),
        in_specs=[pl.BlockSpec(memory_space=pltpu.VMEM)] * 5,
        out_specs=pl.BlockSpec(memory_space=pltpu.VMEM),
        scratch_shapes=[
            pltpu.VMEM((N_DEV, SQ, D), jnp.bfloat16),
            pltpu.VMEM((SQ, D), jnp.float32),
            pltpu.VMEM((SQ, D), jnp.bfloat16),
            pltpu.VMEM((SQ, D), jnp.bfloat16),
            pltpu.VMEM((SQ, D), jnp.bfloat16),
            pltpu.VMEM((SQ, D), jnp.bfloat16),
            pltpu.VMEM((SQ, D), jnp.bfloat16),
            pltpu.VMEM((HA, D), jnp.bfloat16),
            pltpu.VMEM((HA, D), jnp.bfloat16),
            pltpu.VMEM((SQ, D), jnp.bfloat16),
            pltpu.VMEM((D, D), jnp.bfloat16),
            pltpu.VMEM((D, D), jnp.bfloat16),
            pltpu.VMEM((D, D), jnp.bfloat16),
            pltpu.VMEM((D, D), jnp.bfloat16),
            pltpu.SemaphoreType.DMA((12,)),
            pltpu.SemaphoreType.DMA((12,)),
        ],
        compiler_params=pltpu.CompilerParams(collective_id=0),
    )(x, Wq, Wk, Wv, Wo)
